# baseline (speedup 1.0000x reference)
import functools

import jax
import jax.numpy as jnp
from jax import lax
from jax.experimental import pallas as pl
from jax.experimental.pallas import tpu as pltpu
from jax.experimental.pallas import tpu_sc as plsc

_BATCH = 16384
_EMBED_DIM = 128
_HIDDEN = 256
_N_ACTIONS = 18

_NC = 2
_NS = 16
_NW = _NC * _NS
_LANES = 128
_ROWS_PER_W = _BATCH // _NW
_CHUNKS = _ROWS_PER_W // _LANES
_IDX_ROWS = _BATCH // _LANES


def _make_sc_gather_body(chunks):
    def _sc_gather_body(embed_hbm, idx_hbm, out_hbm, idx_v, buf_v,
                        gsem, wsem):
        wid = lax.axis_index("s") * _NC + lax.axis_index("c")
        base = wid * chunks
        pltpu.sync_copy(idx_hbm.at[pl.ds(base, chunks)], idx_v)
        for j in range(chunks):
            pltpu.async_copy(embed_hbm.at[idx_v.at[j]], buf_v.at[j], gsem)
        for j in range(chunks):
            pltpu.make_async_copy(embed_hbm.at[idx_v.at[j]], buf_v.at[j],
                                  gsem).wait()
            pltpu.async_copy(buf_v.at[j], out_hbm.at[base + j], wsem)
        for j in range(chunks):
            pltpu.make_async_copy(buf_v.at[j], out_hbm.at[base + j],
                                  wsem).wait()
    return _sc_gather_body


def _sc_gather(embed, idx2d):
    rows = idx2d.shape[0]
    chunks = rows // _NW
    mesh = plsc.VectorSubcoreMesh(core_axis_name="c", subcore_axis_name="s",
                                  num_cores=_NC, num_subcores=_NS)
    f = functools.partial(
        pl.kernel,
        out_type=jax.ShapeDtypeStruct((rows, _LANES, _EMBED_DIM),
                                      jnp.float32),
        mesh=mesh,
        scratch_types=[
            pltpu.VMEM((chunks, _LANES), jnp.int32),
            pltpu.VMEM((chunks, _LANES, _EMBED_DIM), jnp.float32),
            pltpu.SemaphoreType.DMA,
            pltpu.SemaphoreType.DMA,
        ],
    )(_make_sc_gather_body(chunks))
    return f(embed, idx2d)


def _mlp_body(x_ref, w1_ref, b1_ref, wc_ref, bc_ref, lp_ref, v_ref):
    x = x_ref[...]
    zt = lax.dot_general(w1_ref[...], x, (((1,), (1,)), ((), ())),
                         preferred_element_type=jnp.float32)
    zt = zt + b1_ref[...]
    ht = zt * jax.nn.sigmoid(zt)
    cat = lax.dot_general(wc_ref[...], ht, (((1,), (0,)), ((), ())),
                          preferred_element_type=jnp.float32)
    cat = cat + bc_ref[...]
    logits = cat[:_N_ACTIONS, :]
    m = jnp.max(logits, axis=0, keepdims=True)
    e = jnp.exp(logits - m)
    s = jnp.sum(e, axis=0, keepdims=True)
    lp_ref[...] = (logits - m - jnp.log(s)).T
    v_ref[...] = cat[_N_ACTIONS:_N_ACTIONS + 1, :].T


def _mlp(x, W1, b1c, Wc, bcc, block_b=2048):
    rows = x.shape[0]
    grid = (rows // block_b,)
    return pl.pallas_call(
        _mlp_body,
        grid=grid,
        in_specs=[
            pl.BlockSpec((block_b, _EMBED_DIM), lambda i: (i, 0)),
            pl.BlockSpec((_HIDDEN, _EMBED_DIM), lambda i: (0, 0)),
            pl.BlockSpec((_HIDDEN, 1), lambda i: (0, 0)),
            pl.BlockSpec((_N_ACTIONS + 1, _HIDDEN), lambda i: (0, 0)),
            pl.BlockSpec((_N_ACTIONS + 1, 1), lambda i: (0, 0)),
        ],
        out_specs=[
            pl.BlockSpec((block_b, _N_ACTIONS), lambda i: (i, 0)),
            pl.BlockSpec((block_b, 1), lambda i: (i, 0)),
        ],
        out_shape=[
            jax.ShapeDtypeStruct((rows, _N_ACTIONS), jnp.float32),
            jax.ShapeDtypeStruct((rows, 1), jnp.float32),
        ],
    )(x, W1, b1c, Wc, bcc)


def kernel(inputs, embed, W1, b1, Wv, bv, Wp, bp):
    idx2d = inputs.astype(jnp.int32).reshape(_IDX_ROWS, _LANES)
    Wc = jnp.concatenate([Wp, Wv], axis=0)
    bcc = jnp.concatenate([bp, bv], axis=0).reshape(_N_ACTIONS + 1, 1)
    b1c = b1.reshape(_HIDDEN, 1)
    n_pipe = 2
    rows = _IDX_ROWS // n_pipe
    xs = [_sc_gather(embed, lax.slice(idx2d, (k * rows, 0),
                                      ((k + 1) * rows, _LANES)))
          for k in range(n_pipe)]
    outs = [_mlp(x.reshape(rows * _LANES, _EMBED_DIM), W1, b1c, Wc, bcc)
            for x in xs]
    log_probs = jnp.concatenate([o[0] for o in outs], axis=0)
    value = jnp.concatenate([o[1] for o in outs], axis=0)
    return (log_probs, value)


# speedup vs baseline: 1.0298x; 1.0298x over previous
import functools

import jax
import jax.numpy as jnp
from jax import lax
from jax.experimental import pallas as pl
from jax.experimental.pallas import tpu as pltpu
from jax.experimental.pallas import tpu_sc as plsc

_BATCH = 16384
_EMBED_DIM = 128
_HIDDEN = 256
_N_ACTIONS = 18

_NC = 2
_NS = 16
_NW = _NC * _NS
_LANES = 128
_ROWS_PER_W = _BATCH // _NW
_CHUNKS = _ROWS_PER_W // _LANES
_IDX_ROWS = _BATCH // _LANES


def _make_sc_gather_body(chunks):
    def _sc_gather_body(embed_hbm, idx_hbm, out_hbm, idx_v, buf_v,
                        gsem, wsem):
        wid = lax.axis_index("s") * _NC + lax.axis_index("c")
        base = wid * chunks
        pltpu.sync_copy(idx_hbm.at[pl.ds(base, chunks)], idx_v)
        for j in range(chunks):
            pltpu.async_copy(embed_hbm.at[idx_v.at[j]], buf_v.at[j], gsem)
        for j in range(chunks):
            pltpu.make_async_copy(embed_hbm.at[idx_v.at[j]], buf_v.at[j],
                                  gsem).wait()
            pltpu.async_copy(buf_v.at[j], out_hbm.at[base + j], wsem)
        for j in range(chunks):
            pltpu.make_async_copy(buf_v.at[j], out_hbm.at[base + j],
                                  wsem).wait()
    return _sc_gather_body


def _sc_gather(embed, idx2d):
    rows = idx2d.shape[0]
    chunks = rows // _NW
    mesh = plsc.VectorSubcoreMesh(core_axis_name="c", subcore_axis_name="s",
                                  num_cores=_NC, num_subcores=_NS)
    f = functools.partial(
        pl.kernel,
        out_type=jax.ShapeDtypeStruct((rows, _LANES, _EMBED_DIM),
                                      jnp.float32),
        mesh=mesh,
        scratch_types=[
            pltpu.VMEM((chunks, _LANES), jnp.int32),
            pltpu.VMEM((chunks, _LANES, _EMBED_DIM), jnp.float32),
            pltpu.SemaphoreType.DMA,
            pltpu.SemaphoreType.DMA,
        ],
    )(_make_sc_gather_body(chunks))
    return f(embed, idx2d)


def _mlp_body(x_ref, w1_ref, b1_ref, wc_ref, bc_ref, lp_ref, v_ref):
    x = x_ref[...]
    zt = lax.dot_general(w1_ref[...], x, (((1,), (1,)), ((), ())),
                         preferred_element_type=jnp.float32)
    zt = zt + b1_ref[...]
    ht = zt * jax.nn.sigmoid(zt)
    cat = lax.dot_general(wc_ref[...], ht, (((1,), (0,)), ((), ())),
                          preferred_element_type=jnp.float32)
    cat = cat + bc_ref[...]
    logits = cat[:_N_ACTIONS, :]
    m = jnp.max(logits, axis=0, keepdims=True)
    e = jnp.exp(logits - m)
    s = jnp.sum(e, axis=0, keepdims=True)
    lp_ref[...] = (logits - m - jnp.log(s)).T
    v_ref[...] = cat[_N_ACTIONS:_N_ACTIONS + 1, :].T


def _mlp(x, W1, b1c, Wc, bcc, block_b=2048):
    rows = x.shape[0]
    grid = (rows // block_b,)
    return pl.pallas_call(
        _mlp_body,
        grid=grid,
        in_specs=[
            pl.BlockSpec((block_b, _EMBED_DIM), lambda i: (i, 0)),
            pl.BlockSpec((_HIDDEN, _EMBED_DIM), lambda i: (0, 0)),
            pl.BlockSpec((_HIDDEN, 1), lambda i: (0, 0)),
            pl.BlockSpec((_N_ACTIONS + 1, _HIDDEN), lambda i: (0, 0)),
            pl.BlockSpec((_N_ACTIONS + 1, 1), lambda i: (0, 0)),
        ],
        out_specs=[
            pl.BlockSpec((block_b, _N_ACTIONS), lambda i: (i, 0)),
            pl.BlockSpec((block_b, 1), lambda i: (i, 0)),
        ],
        out_shape=[
            jax.ShapeDtypeStruct((rows, _N_ACTIONS), jnp.float32),
            jax.ShapeDtypeStruct((rows, 1), jnp.float32),
        ],
    )(x, W1, b1c, Wc, bcc)


def kernel(inputs, embed, W1, b1, Wv, bv, Wp, bp):
    idx2d = inputs.astype(jnp.int32).reshape(_IDX_ROWS, _LANES)
    Wc = jnp.concatenate([Wp, Wv], axis=0)
    bcc = jnp.concatenate([bp, bv], axis=0).reshape(_N_ACTIONS + 1, 1)
    b1c = b1.reshape(_HIDDEN, 1)
    x = _sc_gather(embed, idx2d).reshape(_BATCH, _EMBED_DIM)
    log_probs, value = _mlp(x, W1, b1c, Wc, bcc, block_b=4096)
    return (log_probs, value)


# speedup vs baseline: 1.0482x; 1.0179x over previous
import functools

import jax
import jax.numpy as jnp
from jax import lax
from jax.experimental import pallas as pl
from jax.experimental.pallas import tpu as pltpu
from jax.experimental.pallas import tpu_sc as plsc

_BATCH = 16384
_EMBED_DIM = 128
_HIDDEN = 256
_N_ACTIONS = 18

_NC = 2
_NS = 16
_NW = _NC * _NS
_LANES = 128
_ROWS_PER_W = _BATCH // _NW
_CHUNKS = _ROWS_PER_W // _LANES
_IDX_ROWS = _BATCH // _LANES


def _make_sc_gather_body(chunks):
    def _sc_gather_body(embed_hbm, idx_hbm, out_hbm, idx_v, buf_v,
                        gsem, wsem):
        wid = lax.axis_index("s") * _NC + lax.axis_index("c")
        base = wid * chunks
        pltpu.sync_copy(idx_hbm.at[pl.ds(base, chunks)], idx_v)
        for j in range(chunks):
            pltpu.async_copy(embed_hbm.at[idx_v.at[j]], buf_v.at[j], gsem)
        for j in range(chunks):
            pltpu.make_async_copy(embed_hbm.at[idx_v.at[j]], buf_v.at[j],
                                  gsem).wait()
            pltpu.async_copy(buf_v.at[j], out_hbm.at[base + j], wsem)
        for j in range(chunks):
            pltpu.make_async_copy(buf_v.at[j], out_hbm.at[base + j],
                                  wsem).wait()
    return _sc_gather_body


def _sc_gather(embed, idx2d):
    rows = idx2d.shape[0]
    chunks = rows // _NW
    mesh = plsc.VectorSubcoreMesh(core_axis_name="c", subcore_axis_name="s",
                                  num_cores=_NC, num_subcores=_NS)
    f = functools.partial(
        pl.kernel,
        out_type=jax.ShapeDtypeStruct((rows, _LANES, _EMBED_DIM),
                                      jnp.float32),
        mesh=mesh,
        scratch_types=[
            pltpu.VMEM((chunks, _LANES), jnp.int32),
            pltpu.VMEM((chunks, _LANES, _EMBED_DIM), jnp.float32),
            pltpu.SemaphoreType.DMA,
            pltpu.SemaphoreType.DMA,
        ],
    )(_make_sc_gather_body(chunks))
    return f(embed, idx2d)


def _mlp_body(x_ref, w1_ref, b1_ref, wc_ref, bc_ref, lp_ref, v_ref):
    x = x_ref[...]
    zt = lax.dot_general(w1_ref[...], x, (((1,), (1,)), ((), ())),
                         preferred_element_type=jnp.float32)
    zt = zt + b1_ref[...]
    ht = zt * (0.5 * jnp.tanh(0.5 * zt) + 0.5)
    cat = lax.dot_general(wc_ref[...], ht, (((1,), (0,)), ((), ())),
                          preferred_element_type=jnp.float32)
    cat = cat + bc_ref[...]
    logits = cat[:_N_ACTIONS, :]
    m = jnp.max(logits, axis=0, keepdims=True)
    e = jnp.exp(logits - m)
    s = jnp.sum(e, axis=0, keepdims=True)
    lp_ref[...] = (logits - m - jnp.log(s)).T
    v_ref[...] = cat[_N_ACTIONS:_N_ACTIONS + 1, :].T


def _mlp(x, W1, b1c, Wc, bcc, block_b=2048):
    rows = x.shape[0]
    grid = (rows // block_b,)
    return pl.pallas_call(
        _mlp_body,
        grid=grid,
        in_specs=[
            pl.BlockSpec((block_b, _EMBED_DIM), lambda i: (i, 0)),
            pl.BlockSpec((_HIDDEN, _EMBED_DIM), lambda i: (0, 0)),
            pl.BlockSpec((_HIDDEN, 1), lambda i: (0, 0)),
            pl.BlockSpec((_N_ACTIONS + 1, _HIDDEN), lambda i: (0, 0)),
            pl.BlockSpec((_N_ACTIONS + 1, 1), lambda i: (0, 0)),
        ],
        out_specs=[
            pl.BlockSpec((block_b, _N_ACTIONS), lambda i: (i, 0)),
            pl.BlockSpec((block_b, 1), lambda i: (i, 0)),
        ],
        out_shape=[
            jax.ShapeDtypeStruct((rows, _N_ACTIONS), jnp.float32),
            jax.ShapeDtypeStruct((rows, 1), jnp.float32),
        ],
    )(x, W1, b1c, Wc, bcc)


def kernel(inputs, embed, W1, b1, Wv, bv, Wp, bp):
    idx2d = inputs.astype(jnp.int32).reshape(_IDX_ROWS, _LANES)
    Wc = jnp.concatenate([Wp, Wv], axis=0)
    bcc = jnp.concatenate([bp, bv], axis=0).reshape(_N_ACTIONS + 1, 1)
    b1c = b1.reshape(_HIDDEN, 1)
    x = _sc_gather(embed, idx2d).reshape(_BATCH, _EMBED_DIM)
    log_probs, value = _mlp(x, W1, b1c, Wc, bcc, block_b=4096)
    return (log_probs, value)
